# baseline (speedup 1.0000x reference)
import functools

import jax
import jax.numpy as jnp
from jax import lax
from jax.experimental import pallas as pl
from jax.experimental.pallas import tpu as pltpu
from jax.experimental.pallas import tpu_sc as plsc

_LANES = 16


def _params_body(w1_ref, w2_ref, w3_ref, b1_ref, b2_ref, b3_ref,
                 weff_ref, c_ref):
    dn = (((1,), (0,)), ((), ()))
    v2 = lax.dot_general(w2_ref[...], w3_ref[...], dn,
                         preferred_element_type=jnp.float32)
    weff_ref[...] = lax.dot_general(w1_ref[...], v2, dn,
                                    preferred_element_type=jnp.float32)
    c1 = lax.dot_general(b1_ref[...], v2, dn,
                         preferred_element_type=jnp.float32)
    c2 = lax.dot_general(b2_ref[...], w3_ref[...], dn,
                         preferred_element_type=jnp.float32)
    c_ref[...] = c1 + c2 + b3_ref[...]


def _collapse_mlp(W1, b1, W2, b2, W3, b3):
    mk, hid = W1.shape
    weff, c = pl.pallas_call(
        _params_body,
        out_shape=[
            jax.ShapeDtypeStruct((mk, 1), jnp.float32),
            jax.ShapeDtypeStruct((1, 1), jnp.float32),
        ],
    )(W1, W2, W3, b1.reshape(1, hid), b2.reshape(1, hid), b3.reshape(1, 1))
    return weff, c


def _finale_body(t_ref, e1_ref, c_ref, y_ref):
    z = (jnp.sum(t_ref[...], axis=1, keepdims=True)
         + jnp.sum(e1_ref[...], axis=1, keepdims=True) + c_ref[0, 0])
    y_ref[...] = jax.nn.sigmoid(z)


def _finale(tb, e1b, c, B):
    return pl.pallas_call(
        _finale_body,
        out_shape=jax.ShapeDtypeStruct((B, 1), jnp.float32),
    )(tb, e1b, c)


def _make_sc_fm(B, M, VOCAB, K):
    NW = 32
    SPW = B // NW
    RPW = SPW * M
    CH_S = 4
    CH_R = CH_S * M
    NCH = SPW // CH_S
    KV = K // _LANES

    mesh = plsc.VectorSubcoreMesh(core_axis_name="c", subcore_axis_name="s",
                                  num_cores=2, num_subcores=16)

    @functools.partial(
        pl.kernel,
        out_type=[
            jax.ShapeDtypeStruct((B * _LANES,), jnp.float32),
            jax.ShapeDtypeStruct((B * M,), jnp.float32),
        ],
        mesh=mesh,
        compiler_params=pltpu.CompilerParams(use_tc_tiling_on_sc=False),
        scratch_types=[
            pltpu.VMEM((RPW,), jnp.int32),
            pltpu.VMEM((RPW,), jnp.int32),
            pltpu.VMEM((RPW,), jnp.float32),
            pltpu.VMEM((CH_R, K), jnp.float32),
            pltpu.VMEM((M * K,), jnp.float32),
            pltpu.VMEM((SPW * _LANES,), jnp.float32),
            pltpu.SemaphoreType.DMA,
            pltpu.SemaphoreType.DMA,
        ],
    )
    def sc_fm(x_hbm, e1_hbm, e2_hbm, w_hbm, t_out, e1_out,
              xb, idxf, e1b, rows, wv, tb, sem, sem2):
        wid = lax.axis_index("s") * 2 + lax.axis_index("c")
        base_r = wid * RPW
        base_s = wid * SPW
        iota = lax.iota(jnp.int32, _LANES)

        pltpu.sync_copy(x_hbm.at[pl.ds(base_r, RPW)], xb)
        pltpu.sync_copy(w_hbm, wv)

        def build(i, _):
            j0 = i * _LANES
            jv = iota + j0
            idxf[pl.ds(j0, _LANES)] = xb[pl.ds(j0, _LANES)] + (jv % M) * VOCAB
            return 0

        lax.fori_loop(0, RPW // _LANES, build, 0, unroll=False)

        zeros = jnp.zeros((_LANES,), jnp.float32)

        def chunk_body(ch, _):
            r0 = ch * CH_R
            cp1 = pltpu.async_copy(e2_hbm.at[idxf.at[pl.ds(r0, CH_R)]],
                                   rows, sem)
            cp2 = pltpu.async_copy(e1_hbm.at[idxf.at[pl.ds(r0, CH_R)]],
                                   e1b.at[pl.ds(r0, CH_R)], sem2)
            cp1.wait()
            cp2.wait()
            for sl in range(CH_S):
                def macc(m, carry):
                    sacc, qv, dv = carry
                    r = sl * M + m
                    sacc2, qv2, dv2 = [], qv, dv
                    for k in range(KV):
                        v = rows[r, pl.ds(k * _LANES, _LANES)]
                        w = wv[pl.ds(m * K + k * _LANES, _LANES)]
                        sacc2.append(sacc[k] + v)
                        qv2 = qv2 + v * v
                        dv2 = dv2 + v * w
                    return tuple(sacc2), qv2, dv2

                sacc, qv, dv = lax.fori_loop(
                    0, M, macc, ((zeros,) * KV, zeros, zeros), unroll=False)
                ssq = zeros
                for k in range(KV):
                    ssq = ssq + sacc[k] * sacc[k]
                sid = ch * CH_S + sl
                tb[pl.ds(sid * _LANES, _LANES)] = 0.5 * (ssq - qv) + dv
            return 0

        lax.fori_loop(0, NCH, chunk_body, 0, unroll=False)

        pltpu.sync_copy(tb, t_out.at[pl.ds(base_s * _LANES, SPW * _LANES)])
        pltpu.sync_copy(e1b, e1_out.at[pl.ds(base_r, RPW)])

    return sc_fm


def kernel(x, E1, E2, W1, b1, W2, b2, W3, b3):
    B, M = x.shape
    _, VOCAB, K = E2.shape
    weff, c = _collapse_mlp(W1, b1, W2, b2, W3, b3)
    sc_fm = _make_sc_fm(B, M, VOCAB, K)
    tb, e1b = sc_fm(
        x.reshape(B * M),
        E1.reshape(M * VOCAB),
        E2.reshape(M * VOCAB, K),
        weff.reshape(M * K),
    )
    return _finale(tb.reshape(B, _LANES), e1b.reshape(B, M), c, B)
